# baseline (speedup 1.0000x reference)
import jax
import jax.numpy as jnp
from jax.experimental import pallas as pl
from jax.experimental.pallas import tpu as pltpu

BK = 1024


def _mm_kernel(w_ref, x_ref, o_ref):
    j = pl.program_id(1)
    w = w_ref[...].astype(jnp.bfloat16)
    xs = x_ref[pl.ds(j * BK, BK), :].astype(jnp.bfloat16)
    acc = jax.lax.dot_general(
        w, xs, (((0,), (0,)), ((), ())),
        preferred_element_type=jnp.float32)

    @pl.when(j == 0)
    def _init():
        o_ref[...] = acc

    @pl.when(j != 0)
    def _accum():
        o_ref[...] += acc


def kernel(in_values, W, bias):
    x = in_values
    if x.ndim == 1:
        x = x.reshape(x.shape[0], 1)
    if x.shape[0] != W.shape[0]:
        x = x.T
    k, m = W.shape
    n = x.shape[1]
    bm = 512
    out = pl.pallas_call(
        _mm_kernel,
        grid=(m // bm, k // BK),
        in_specs=[
            pl.BlockSpec((BK, bm), lambda i, j: (j, i)),
            pl.BlockSpec((k, n), lambda i, j: (0, 0)),
        ],
        out_specs=pl.BlockSpec((bm, n), lambda i, j: (i, 0)),
        out_shape=jax.ShapeDtypeStruct((m, n), jnp.float32),
        compiler_params=pltpu.CompilerParams(
            dimension_semantics=("parallel", "arbitrary"),
            vmem_limit_bytes=120 * 1024 * 1024,
        ),
    )(W, x)
    return out
